# Initial kernel scaffold; baseline (speedup 1.0000x reference)
import jax
import jax.numpy as jnp
from jax.experimental import pallas as pl


def kernel(author_idx, publisher_idx, year_idx, author_table, publisher_table, year_table):
    raise NotImplementedError("write your pallas kernel here")


# speedup vs baseline: 1.7954x; 1.7954x over previous
import functools

import jax
import jax.numpy as jnp
from jax import lax
from jax.experimental import pallas as pl
from jax.experimental.pallas import tpu as pltpu
from jax.experimental.pallas import tpu_sc as plsc

EMBED = 64
NUM_CORES = 2
NUM_SUBCORES = 16
NUM_WORKERS = NUM_CORES * NUM_SUBCORES
CHUNK = 128


def kernel(author_idx, publisher_idx, year_idx, author_table,
           publisher_table, year_table):
    batch = author_idx.shape[0]
    b_per_w = batch // NUM_WORKERS
    n_chunks = b_per_w // CHUNK

    def prep(idx):
        return idx.astype(jnp.int32).reshape(NUM_WORKERS, n_chunks, CHUNK)

    a_idx = prep(author_idx)
    p_idx = prep(publisher_idx)
    y_idx = prep(year_idx)

    mesh = plsc.VectorSubcoreMesh(core_axis_name="c", subcore_axis_name="s")
    row_t = jax.ShapeDtypeStruct((batch, EMBED), jnp.float32)

    @functools.partial(
        pl.kernel,
        mesh=mesh,
        out_type=[row_t, row_t, row_t],
        scratch_types=[
            pltpu.VMEM((n_chunks, CHUNK), jnp.int32),
            pltpu.VMEM((n_chunks, CHUNK), jnp.int32),
            pltpu.VMEM((n_chunks, CHUNK), jnp.int32),
            pltpu.VMEM((b_per_w, EMBED), jnp.float32),
            pltpu.VMEM((b_per_w, EMBED), jnp.float32),
            pltpu.VMEM((b_per_w, EMBED), jnp.float32),
            pltpu.SemaphoreType.DMA,
        ],
        compiler_params=pltpu.CompilerParams(use_tc_tiling_on_sc=False),
    )
    def sc_gather3(a_t, p_t, y_t, ai, pi, yi, out_a, out_p, out_y,
                   ai_v, pi_v, yi_v, ar_v, pr_v, yr_v, sem):
        wid = lax.axis_index("s") * NUM_CORES + lax.axis_index("c")
        base = wid * b_per_w
        pltpu.sync_copy(ai.at[wid], ai_v)
        pltpu.sync_copy(pi.at[wid], pi_v)
        pltpu.sync_copy(yi.at[wid], yi_v)
        copies = []
        for j in range(n_chunks):
            rows = pl.ds(j * CHUNK, CHUNK)
            copies.append(
                pltpu.async_copy(a_t.at[ai_v.at[j]], ar_v.at[rows], sem))
            copies.append(
                pltpu.async_copy(p_t.at[pi_v.at[j]], pr_v.at[rows], sem))
            copies.append(
                pltpu.async_copy(y_t.at[yi_v.at[j]], yr_v.at[rows], sem))
        for c in copies:
            c.wait()
        dst = pl.ds(base, b_per_w)
        pltpu.sync_copy(ar_v, out_a.at[dst])
        pltpu.sync_copy(pr_v, out_p.at[dst])
        pltpu.sync_copy(yr_v, out_y.at[dst])

    out_a, out_p, out_y = sc_gather3(author_table, publisher_table,
                                     year_table, a_idx, p_idx, y_idx)
    return jnp.concatenate((out_a, out_p, out_y), axis=1)


# speedup vs baseline: 2.1758x; 1.2119x over previous
import functools

import jax
import jax.numpy as jnp
from jax import lax
from jax.experimental import pallas as pl
from jax.experimental.pallas import tpu as pltpu
from jax.experimental.pallas import tpu_sc as plsc

EMBED = 64
NUM_CORES = 2
NUM_SUBCORES = 16
NUM_WORKERS = NUM_CORES * NUM_SUBCORES
CHUNK = 128


def kernel(author_idx, publisher_idx, year_idx, author_table,
           publisher_table, year_table):
    batch = author_idx.shape[0]
    b_per_w = batch // NUM_WORKERS
    n_chunks = b_per_w // CHUNK

    def prep(idx):
        return idx.astype(jnp.int32).reshape(NUM_WORKERS, n_chunks, CHUNK)

    a_idx = prep(author_idx)
    p_idx = prep(publisher_idx)
    y_idx = prep(year_idx)

    mesh = plsc.VectorSubcoreMesh(core_axis_name="c", subcore_axis_name="s")

    @functools.partial(
        pl.kernel,
        mesh=mesh,
        out_type=jax.ShapeDtypeStruct((batch, 3 * EMBED), jnp.float32),
        scratch_types=[
            pltpu.VMEM((n_chunks, CHUNK), jnp.int32),
            pltpu.VMEM((n_chunks, CHUNK), jnp.int32),
            pltpu.VMEM((n_chunks, CHUNK), jnp.int32),
            pltpu.VMEM((b_per_w, EMBED), jnp.float32),
            pltpu.VMEM((b_per_w, EMBED), jnp.float32),
            pltpu.VMEM((b_per_w, EMBED), jnp.float32),
            pltpu.SemaphoreType.DMA,
        ],
        compiler_params=pltpu.CompilerParams(use_tc_tiling_on_sc=False),
    )
    def sc_gather3(a_t, p_t, y_t, ai, pi, yi, out,
                   ai_v, pi_v, yi_v, ar_v, pr_v, yr_v, sem):
        wid = lax.axis_index("s") * NUM_CORES + lax.axis_index("c")
        base = wid * b_per_w
        pltpu.sync_copy(ai.at[wid], ai_v)
        pltpu.sync_copy(pi.at[wid], pi_v)
        pltpu.sync_copy(yi.at[wid], yi_v)
        copies = []
        for j in range(n_chunks):
            rows = pl.ds(j * CHUNK, CHUNK)
            copies.append(
                pltpu.async_copy(a_t.at[ai_v.at[j]], ar_v.at[rows], sem))
            copies.append(
                pltpu.async_copy(p_t.at[pi_v.at[j]], pr_v.at[rows], sem))
            copies.append(
                pltpu.async_copy(y_t.at[yi_v.at[j]], yr_v.at[rows], sem))
        for c in copies:
            c.wait()
        dst = pl.ds(base, b_per_w)
        pltpu.sync_copy(ar_v, out.at[dst, pl.ds(0, EMBED)])
        pltpu.sync_copy(pr_v, out.at[dst, pl.ds(EMBED, EMBED)])
        pltpu.sync_copy(yr_v, out.at[dst, pl.ds(2 * EMBED, EMBED)])

    return sc_gather3(author_table, publisher_table, year_table,
                      a_idx, p_idx, y_idx)
